# baseline (speedup 1.0000x reference)
import jax
import jax.numpy as jnp
from jax import lax
from jax.experimental import pallas as pl
from jax.experimental.pallas import tpu as pltpu
from jax.experimental.pallas import tpu_sc as plsc

D_MODEL = 768
D_FF = 3072
N_EXPERTS = 8
N_TOKENS = 2048

BT = 256
SLOTS = ((2 * N_TOKENS + N_EXPERTS * (BT - 1) + BT - 1) // BT) * BT
NT = SLOTS // BT
SBW = 128


def _cumsum_rows(c, n):
    k = 1
    while k < n:
        shifted = jnp.concatenate(
            [jnp.zeros((k, c.shape[1]), c.dtype), c[: n - k]], axis=0)
        c = c + shifted
        k *= 2
    return c


def _gate_body(x_ref, wg_ref, bg_ref, slots_ref, counts_ref, sbc_ref):
    logits = jnp.dot(x_ref[...], wg_ref[...],
                     preferred_element_type=jnp.float32) + bg_ref[0]
    m = jnp.max(logits, axis=1, keepdims=True)
    p = jnp.exp(logits - m)
    probs = p / jnp.sum(p, axis=1, keepdims=True)
    iota = jax.lax.broadcasted_iota(jnp.int32, probs.shape, 1)
    m1 = jnp.max(probs, axis=1, keepdims=True)
    e1 = jnp.min(jnp.where(probs == m1, iota, N_EXPERTS), axis=1, keepdims=True)
    p2 = jnp.where(iota == e1, -jnp.inf, probs)
    m2 = jnp.max(p2, axis=1, keepdims=True)
    e2 = jnp.min(jnp.where(p2 == m2, iota, N_EXPERTS), axis=1, keepdims=True)

    oh1 = (iota == e1).astype(jnp.int32)
    oh2 = (iota == e2).astype(jnp.int32)
    c1 = _cumsum_rows(oh1, N_TOKENS)
    c2 = _cumsum_rows(oh2, N_TOKENS)
    tot1 = c1[N_TOKENS - 1:N_TOKENS]
    counts = tot1 + c2[N_TOKENS - 1:N_TOKENS]
    rank1 = jnp.sum((c1 - 1) * oh1, axis=1, keepdims=True)
    rank2 = jnp.sum((c2 - 1 + tot1) * oh2, axis=1, keepdims=True)

    padded = lax.shift_left(
        lax.shift_right_logical(counts + (BT - 1), BT.bit_length() - 1),
        BT.bit_length() - 1)
    inc = padded
    k = 1
    while k < N_EXPERTS:
        inc = inc + jnp.concatenate(
            [jnp.zeros((1, k), jnp.int32), inc[:, : N_EXPERTS - k]], axis=1)
        k *= 2
    bases = inc - padded

    slot1 = rank1 + jnp.sum(oh1 * bases, axis=1, keepdims=True)
    slot2 = rank2 + jnp.sum(oh2 * bases, axis=1, keepdims=True)
    slots_ref[:, 0:1] = slot1
    slots_ref[:, 1:2] = slot2
    counts_ref[...] = counts
    ones = jnp.ones((1, SBW), jnp.float32)
    sbc_ref[0] = m1 * ones
    sbc_ref[1] = m2 * ones


def _gate(x2d, Wg, bg2):
    return pl.pallas_call(
        _gate_body,
        out_shape=[
            jax.ShapeDtypeStruct((N_TOKENS, 2), jnp.int32),
            jax.ShapeDtypeStruct((1, N_EXPERTS), jnp.int32),
            jax.ShapeDtypeStruct((2, N_TOKENS, SBW), jnp.float32),
        ],
    )(x2d, Wg, bg2)


TPW = N_TOKENS // 32


def _dispatch_body(x_hbm, slots_hbm, sbc_hbm, xd_hbm, ssc_hbm,
                   idx_v, rows_v, srow_v, sem):
    core = lax.axis_index("c")
    sub = lax.axis_index("s")
    wid = sub * 2 + core
    base = wid * TPW
    for ci in range(TPW // 16):
        t0 = pl.multiple_of(base + ci * 16, 16)
        pltpu.sync_copy(x_hbm.at[pl.ds(t0, 16)], rows_v)
        for k in range(2):
            pltpu.sync_copy(slots_hbm.at[k, pl.ds(t0, 16)], idx_v)
            pltpu.async_copy(rows_v, xd_hbm.at[idx_v], sem).wait()
            pltpu.sync_copy(sbc_hbm.at[k, pl.ds(t0, 16)], srow_v)
            pltpu.async_copy(srow_v, ssc_hbm.at[idx_v], sem).wait()


def _dispatch(x2d, slots_t, sbc):
    mesh = plsc.VectorSubcoreMesh(core_axis_name="c", subcore_axis_name="s")
    f = pl.kernel(
        _dispatch_body,
        mesh=mesh,
        out_type=[
            jax.ShapeDtypeStruct((SLOTS, D_MODEL), jnp.float32),
            jax.ShapeDtypeStruct((SLOTS, SBW), jnp.float32),
        ],
        scratch_types=[
            pltpu.VMEM((16,), jnp.int32),
            pltpu.VMEM((16, D_MODEL), jnp.float32),
            pltpu.VMEM((16, SBW), jnp.float32),
            pltpu.SemaphoreType.DMA,
        ],
    )
    return f(x2d, slots_t, sbc)


def _ffn_body(te_ref, tr_ref, xd_ref, w1_ref, b1_ref, w2_ref, b2_ref, s_ref,
              yd_ref):
    t = pl.program_id(0)
    rows = tr_ref[t]

    @pl.when(rows > 0)
    def _():
        xg = xd_ref[...].astype(jnp.bfloat16)
        h = jnp.dot(xg, w1_ref[0].astype(jnp.bfloat16),
                    preferred_element_type=jnp.float32) + b1_ref[0, 0]
        h = h * jax.nn.sigmoid(h)
        y = jnp.dot(h.astype(jnp.bfloat16), w2_ref[0].astype(jnp.bfloat16),
                    preferred_element_type=jnp.float32) + b2_ref[0, 0]
        yd_ref[...] = y * s_ref[:, 0:1]


def _ffn(xd, W1, b1r, W2, b2r, ssc, tile_expert, tile_rows):
    grid_spec = pltpu.PrefetchScalarGridSpec(
        num_scalar_prefetch=2,
        grid=(NT,),
        in_specs=[
            pl.BlockSpec((BT, D_MODEL), lambda t, te, tr: (t, 0)),
            pl.BlockSpec((1, D_MODEL, D_FF), lambda t, te, tr: (te[t], 0, 0)),
            pl.BlockSpec((1, 1, D_FF), lambda t, te, tr: (te[t], 0, 0)),
            pl.BlockSpec((1, D_FF, D_MODEL), lambda t, te, tr: (te[t], 0, 0)),
            pl.BlockSpec((1, 1, D_MODEL), lambda t, te, tr: (te[t], 0, 0)),
            pl.BlockSpec((BT, SBW), lambda t, te, tr: (t, 0)),
        ],
        out_specs=pl.BlockSpec((BT, D_MODEL), lambda t, te, tr: (t, 0)),
    )
    return pl.pallas_call(
        _ffn_body,
        grid_spec=grid_spec,
        out_shape=jax.ShapeDtypeStruct((SLOTS, D_MODEL), jnp.float32),
    )(tile_expert, tile_rows, xd, W1, b1r, W2, b2r, ssc)


def _combine_body(yd_hbm, slots_hbm, out_hbm, ia_v, ib_v, ra_v, rb_v, sem):
    core = lax.axis_index("c")
    sub = lax.axis_index("s")
    wid = sub * 2 + core
    base = wid * TPW
    for ci in range(TPW // 16):
        t0 = pl.multiple_of(base + ci * 16, 16)
        pltpu.sync_copy(slots_hbm.at[0, pl.ds(t0, 16)], ia_v)
        pltpu.sync_copy(slots_hbm.at[1, pl.ds(t0, 16)], ib_v)
        pltpu.async_copy(yd_hbm.at[ia_v], ra_v, sem).wait()
        pltpu.async_copy(yd_hbm.at[ib_v], rb_v, sem).wait()
        for r in range(16):
            def _add(c, _, r=r):
                ra_v[r, pl.ds(c * 16, 16)] = (ra_v[r, pl.ds(c * 16, 16)]
                                              + rb_v[r, pl.ds(c * 16, 16)])
                return 0
            lax.fori_loop(0, D_MODEL // 16, _add, 0)
        pltpu.sync_copy(ra_v, out_hbm.at[pl.ds(t0, 16)])


def _combine(yd, slots_t):
    mesh = plsc.VectorSubcoreMesh(core_axis_name="c", subcore_axis_name="s")
    f = pl.kernel(
        _combine_body,
        mesh=mesh,
        out_type=jax.ShapeDtypeStruct((N_TOKENS, D_MODEL), jnp.float32),
        scratch_types=[
            pltpu.VMEM((16,), jnp.int32),
            pltpu.VMEM((16,), jnp.int32),
            pltpu.VMEM((16, D_MODEL), jnp.float32),
            pltpu.VMEM((16, D_MODEL), jnp.float32),
            pltpu.SemaphoreType.DMA,
        ],
    )
    return f(yd, slots_t)


def kernel(x, Wg, bg, W1, b1, W2, b2):
    x2d = x.reshape(-1, D_MODEL)
    bg2 = bg.reshape(1, N_EXPERTS)
    b1r = b1.reshape(N_EXPERTS, 1, D_FF)
    b2r = b2.reshape(N_EXPERTS, 1, D_MODEL)

    slots, counts, sbc = _gate(x2d, Wg, bg2)
    slots_t = slots.T
    counts = counts.reshape(N_EXPERTS)

    ntiles = (counts + BT - 1) // BT
    padded = ntiles * BT
    starts = jnp.concatenate([jnp.zeros((1,), jnp.int32),
                              jnp.cumsum(padded)[:-1].astype(jnp.int32)])
    tile_expert = jnp.repeat(jnp.arange(N_EXPERTS, dtype=jnp.int32), ntiles,
                             total_repeat_length=NT)
    tstart = jnp.arange(NT, dtype=jnp.int32) * BT
    local = tstart - starts[tile_expert]
    tile_rows = jnp.clip(counts[tile_expert] - local, 0, BT).astype(jnp.int32)

    xd, ssc = _dispatch(x2d, slots_t, sbc)
    yd = _ffn(xd, W1, b1r, W2, b2r, ssc, tile_expert, tile_rows)
    out = _combine(yd, slots_t)
    return out.reshape(x.shape)


# speedup vs baseline: 1.0796x; 1.0796x over previous
import jax
import jax.numpy as jnp
from jax import lax
from jax.experimental import pallas as pl
from jax.experimental.pallas import tpu as pltpu
from jax.experimental.pallas import tpu_sc as plsc

D_MODEL = 768
D_FF = 3072
N_EXPERTS = 8
N_TOKENS = 2048

BT = 512
SLOTS = ((2 * N_TOKENS + N_EXPERTS * (BT - 1) + BT - 1) // BT) * BT
NT = SLOTS // BT
SBW = 128


def _cumsum_rows(c, n):
    k = 1
    while k < n:
        shifted = jnp.concatenate(
            [jnp.zeros((k, c.shape[1]), c.dtype), c[: n - k]], axis=0)
        c = c + shifted
        k *= 2
    return c


def _gate_body(x_ref, wg_ref, bg_ref, slots_ref, counts_ref, sbc_ref):
    logits = jnp.dot(x_ref[...], wg_ref[...],
                     preferred_element_type=jnp.float32) + bg_ref[0]
    m = jnp.max(logits, axis=1, keepdims=True)
    p = jnp.exp(logits - m)
    probs = p / jnp.sum(p, axis=1, keepdims=True)
    iota = jax.lax.broadcasted_iota(jnp.int32, probs.shape, 1)
    m1 = jnp.max(probs, axis=1, keepdims=True)
    e1 = jnp.min(jnp.where(probs == m1, iota, N_EXPERTS), axis=1, keepdims=True)
    p2 = jnp.where(iota == e1, -jnp.inf, probs)
    m2 = jnp.max(p2, axis=1, keepdims=True)
    e2 = jnp.min(jnp.where(p2 == m2, iota, N_EXPERTS), axis=1, keepdims=True)

    oh1 = (iota == e1).astype(jnp.int32)
    oh2 = (iota == e2).astype(jnp.int32)
    c1 = _cumsum_rows(oh1, N_TOKENS)
    c2 = _cumsum_rows(oh2, N_TOKENS)
    tot1 = c1[N_TOKENS - 1:N_TOKENS]
    counts = tot1 + c2[N_TOKENS - 1:N_TOKENS]
    rank1 = jnp.sum((c1 - 1) * oh1, axis=1, keepdims=True)
    rank2 = jnp.sum((c2 - 1 + tot1) * oh2, axis=1, keepdims=True)

    padded = lax.shift_left(
        lax.shift_right_logical(counts + (BT - 1), BT.bit_length() - 1),
        BT.bit_length() - 1)
    inc = padded
    k = 1
    while k < N_EXPERTS:
        inc = inc + jnp.concatenate(
            [jnp.zeros((1, k), jnp.int32), inc[:, : N_EXPERTS - k]], axis=1)
        k *= 2
    bases = inc - padded

    slot1 = rank1 + jnp.sum(oh1 * bases, axis=1, keepdims=True)
    slot2 = rank2 + jnp.sum(oh2 * bases, axis=1, keepdims=True)
    slots_ref[:, 0:1] = slot1
    slots_ref[:, 1:2] = slot2
    counts_ref[...] = counts
    ones = jnp.ones((1, SBW), jnp.float32)
    sbc_ref[0] = m1 * ones
    sbc_ref[1] = m2 * ones


def _gate(x2d, Wg, bg2):
    return pl.pallas_call(
        _gate_body,
        out_shape=[
            jax.ShapeDtypeStruct((N_TOKENS, 2), jnp.int32),
            jax.ShapeDtypeStruct((1, N_EXPERTS), jnp.int32),
            jax.ShapeDtypeStruct((2, N_TOKENS, SBW), jnp.float32),
        ],
    )(x2d, Wg, bg2)


TPW = N_TOKENS // 32


def _dispatch_body(x_hbm, slots_hbm, sbc_hbm, xd_hbm, ssc_hbm,
                   idx_v, rows_v, srow_v, sem):
    core = lax.axis_index("c")
    sub = lax.axis_index("s")
    wid = sub * 2 + core
    base = wid * TPW
    for ci in range(TPW // 16):
        t0 = pl.multiple_of(base + ci * 16, 16)
        pltpu.sync_copy(x_hbm.at[pl.ds(t0, 16)], rows_v)
        for k in range(2):
            pltpu.sync_copy(slots_hbm.at[k, pl.ds(t0, 16)], idx_v)
            pltpu.async_copy(rows_v, xd_hbm.at[idx_v], sem).wait()
            pltpu.sync_copy(sbc_hbm.at[k, pl.ds(t0, 16)], srow_v)
            pltpu.async_copy(srow_v, ssc_hbm.at[idx_v], sem).wait()


def _dispatch(x2d, slots_t, sbc):
    mesh = plsc.VectorSubcoreMesh(core_axis_name="c", subcore_axis_name="s")
    f = pl.kernel(
        _dispatch_body,
        mesh=mesh,
        out_type=[
            jax.ShapeDtypeStruct((SLOTS, D_MODEL), jnp.float32),
            jax.ShapeDtypeStruct((SLOTS, SBW), jnp.float32),
        ],
        scratch_types=[
            pltpu.VMEM((16,), jnp.int32),
            pltpu.VMEM((16, D_MODEL), jnp.float32),
            pltpu.VMEM((16, SBW), jnp.float32),
            pltpu.SemaphoreType.DMA,
        ],
    )
    return f(x2d, slots_t, sbc)


def _ffn_body(te_ref, tr_ref, xd_ref, w1_ref, b1_ref, w2_ref, b2_ref, s_ref,
              yd_ref):
    t = pl.program_id(0)
    rows = tr_ref[t]

    @pl.when(rows > 0)
    def _():
        xg = xd_ref[...].astype(jnp.bfloat16)
        h = jnp.dot(xg, w1_ref[0].astype(jnp.bfloat16),
                    preferred_element_type=jnp.float32) + b1_ref[0, 0]
        h = h * jax.nn.sigmoid(h)
        y = jnp.dot(h.astype(jnp.bfloat16), w2_ref[0].astype(jnp.bfloat16),
                    preferred_element_type=jnp.float32) + b2_ref[0, 0]
        yd_ref[...] = y * s_ref[:, 0:1]


def _ffn(xd, W1, b1r, W2, b2r, ssc, tile_expert, tile_rows):
    grid_spec = pltpu.PrefetchScalarGridSpec(
        num_scalar_prefetch=2,
        grid=(NT,),
        in_specs=[
            pl.BlockSpec((BT, D_MODEL), lambda t, te, tr: (t, 0)),
            pl.BlockSpec((1, D_MODEL, D_FF), lambda t, te, tr: (te[t], 0, 0)),
            pl.BlockSpec((1, 1, D_FF), lambda t, te, tr: (te[t], 0, 0)),
            pl.BlockSpec((1, D_FF, D_MODEL), lambda t, te, tr: (te[t], 0, 0)),
            pl.BlockSpec((1, 1, D_MODEL), lambda t, te, tr: (te[t], 0, 0)),
            pl.BlockSpec((BT, SBW), lambda t, te, tr: (t, 0)),
        ],
        out_specs=pl.BlockSpec((BT, D_MODEL), lambda t, te, tr: (t, 0)),
    )
    return pl.pallas_call(
        _ffn_body,
        grid_spec=grid_spec,
        out_shape=jax.ShapeDtypeStruct((SLOTS, D_MODEL), jnp.float32),
    )(tile_expert, tile_rows, xd, W1, b1r, W2, b2r, ssc)


def _combine_body(yd_hbm, slots_hbm, out_hbm, ia_v, ib_v, ra_v, rb_v, sem):
    core = lax.axis_index("c")
    sub = lax.axis_index("s")
    wid = sub * 2 + core
    base = wid * TPW
    for ci in range(TPW // 16):
        t0 = pl.multiple_of(base + ci * 16, 16)
        pltpu.sync_copy(slots_hbm.at[0, pl.ds(t0, 16)], ia_v)
        pltpu.sync_copy(slots_hbm.at[1, pl.ds(t0, 16)], ib_v)
        pltpu.async_copy(yd_hbm.at[ia_v], ra_v, sem).wait()
        pltpu.async_copy(yd_hbm.at[ib_v], rb_v, sem).wait()
        for r in range(16):
            def _add(c, _, r=r):
                ra_v[r, pl.ds(c * 16, 16)] = (ra_v[r, pl.ds(c * 16, 16)]
                                              + rb_v[r, pl.ds(c * 16, 16)])
                return 0
            lax.fori_loop(0, D_MODEL // 16, _add, 0)
        pltpu.sync_copy(ra_v, out_hbm.at[pl.ds(t0, 16)])


def _combine(yd, slots_t):
    mesh = plsc.VectorSubcoreMesh(core_axis_name="c", subcore_axis_name="s")
    f = pl.kernel(
        _combine_body,
        mesh=mesh,
        out_type=jax.ShapeDtypeStruct((N_TOKENS, D_MODEL), jnp.float32),
        scratch_types=[
            pltpu.VMEM((16,), jnp.int32),
            pltpu.VMEM((16,), jnp.int32),
            pltpu.VMEM((16, D_MODEL), jnp.float32),
            pltpu.VMEM((16, D_MODEL), jnp.float32),
            pltpu.SemaphoreType.DMA,
        ],
    )
    return f(yd, slots_t)


def kernel(x, Wg, bg, W1, b1, W2, b2):
    x2d = x.reshape(-1, D_MODEL)
    bg2 = bg.reshape(1, N_EXPERTS)
    b1r = b1.reshape(N_EXPERTS, 1, D_FF)
    b2r = b2.reshape(N_EXPERTS, 1, D_MODEL)

    slots, counts, sbc = _gate(x2d, Wg, bg2)
    slots_t = slots.T
    counts = counts.reshape(N_EXPERTS)

    ntiles = (counts + BT - 1) // BT
    padded = ntiles * BT
    starts = jnp.concatenate([jnp.zeros((1,), jnp.int32),
                              jnp.cumsum(padded)[:-1].astype(jnp.int32)])
    tile_expert = jnp.repeat(jnp.arange(N_EXPERTS, dtype=jnp.int32), ntiles,
                             total_repeat_length=NT)
    tstart = jnp.arange(NT, dtype=jnp.int32) * BT
    local = tstart - starts[tile_expert]
    tile_rows = jnp.clip(counts[tile_expert] - local, 0, BT).astype(jnp.int32)

    xd, ssc = _dispatch(x2d, slots_t, sbc)
    yd = _ffn(xd, W1, b1r, W2, b2r, ssc, tile_expert, tile_rows)
    out = _combine(yd, slots_t)
    return out.reshape(x.shape)


# speedup vs baseline: 1.1960x; 1.1077x over previous
import jax
import jax.numpy as jnp
from jax import lax
from jax.experimental import pallas as pl
from jax.experimental.pallas import tpu as pltpu
from jax.experimental.pallas import tpu_sc as plsc

D_MODEL = 768
D_FF = 3072
N_EXPERTS = 8
N_TOKENS = 2048

BT = 512
SLOTS = ((2 * N_TOKENS + N_EXPERTS * (BT - 1) + BT - 1) // BT) * BT
NT = SLOTS // BT
SBW = 128


def _cumsum_rows(c, n):
    k = 1
    while k < n:
        shifted = jnp.concatenate(
            [jnp.zeros((k, c.shape[1]), c.dtype), c[: n - k]], axis=0)
        c = c + shifted
        k *= 2
    return c


def _gate_body(x_ref, wg_ref, bg_ref, slots_ref, counts_ref, sbc_ref):
    logits = jnp.dot(x_ref[...], wg_ref[...],
                     preferred_element_type=jnp.float32) + bg_ref[0]
    m = jnp.max(logits, axis=1, keepdims=True)
    p = jnp.exp(logits - m)
    probs = p / jnp.sum(p, axis=1, keepdims=True)
    iota = jax.lax.broadcasted_iota(jnp.int32, probs.shape, 1)
    m1 = jnp.max(probs, axis=1, keepdims=True)
    e1 = jnp.min(jnp.where(probs == m1, iota, N_EXPERTS), axis=1, keepdims=True)
    p2 = jnp.where(iota == e1, -jnp.inf, probs)
    m2 = jnp.max(p2, axis=1, keepdims=True)
    e2 = jnp.min(jnp.where(p2 == m2, iota, N_EXPERTS), axis=1, keepdims=True)

    oh1 = (iota == e1).astype(jnp.int32)
    oh2 = (iota == e2).astype(jnp.int32)
    c1 = _cumsum_rows(oh1, N_TOKENS)
    c2 = _cumsum_rows(oh2, N_TOKENS)
    tot1 = c1[N_TOKENS - 1:N_TOKENS]
    counts = tot1 + c2[N_TOKENS - 1:N_TOKENS]
    rank1 = jnp.sum((c1 - 1) * oh1, axis=1, keepdims=True)
    rank2 = jnp.sum((c2 - 1 + tot1) * oh2, axis=1, keepdims=True)

    padded = lax.shift_left(
        lax.shift_right_logical(counts + (BT - 1), BT.bit_length() - 1),
        BT.bit_length() - 1)
    inc = padded
    k = 1
    while k < N_EXPERTS:
        inc = inc + jnp.concatenate(
            [jnp.zeros((1, k), jnp.int32), inc[:, : N_EXPERTS - k]], axis=1)
        k *= 2
    bases = inc - padded

    slot1 = rank1 + jnp.sum(oh1 * bases, axis=1, keepdims=True)
    slot2 = rank2 + jnp.sum(oh2 * bases, axis=1, keepdims=True)
    slots_ref[:, 0:1] = slot1
    slots_ref[:, 1:2] = slot2
    counts_ref[...] = counts
    ones = jnp.ones((1, SBW), jnp.float32)
    sbc_ref[0] = m1 * ones
    sbc_ref[1] = m2 * ones


def _gate(x2d, Wg, bg2):
    return pl.pallas_call(
        _gate_body,
        out_shape=[
            jax.ShapeDtypeStruct((N_TOKENS, 2), jnp.int32),
            jax.ShapeDtypeStruct((1, N_EXPERTS), jnp.int32),
            jax.ShapeDtypeStruct((2, N_TOKENS, SBW), jnp.float32),
        ],
    )(x2d, Wg, bg2)


TPW = N_TOKENS // 32


def _dispatch_body(x_hbm, slots_hbm, sbc_hbm, xd_hbm, ssc_hbm,
                   ia_v, ib_v, rows_v, sa_v, sb_v, sem):
    core = lax.axis_index("c")
    sub = lax.axis_index("s")
    wid = sub * 2 + core
    t0 = pl.multiple_of(wid * TPW, TPW)
    pltpu.sync_copy(slots_hbm.at[0, pl.ds(t0, TPW)], ia_v)
    pltpu.sync_copy(slots_hbm.at[1, pl.ds(t0, TPW)], ib_v)
    pltpu.sync_copy(sbc_hbm.at[0, pl.ds(t0, TPW)], sa_v)
    pltpu.sync_copy(sbc_hbm.at[1, pl.ds(t0, TPW)], sb_v)
    pltpu.sync_copy(x_hbm.at[pl.ds(t0, TPW)], rows_v)
    c1 = pltpu.async_copy(rows_v, xd_hbm.at[ia_v], sem)
    c2 = pltpu.async_copy(rows_v, xd_hbm.at[ib_v], sem)
    c3 = pltpu.async_copy(sa_v, ssc_hbm.at[ia_v], sem)
    c4 = pltpu.async_copy(sb_v, ssc_hbm.at[ib_v], sem)
    c1.wait()
    c2.wait()
    c3.wait()
    c4.wait()


def _dispatch(x2d, slots_t, sbc):
    mesh = plsc.VectorSubcoreMesh(core_axis_name="c", subcore_axis_name="s")
    f = pl.kernel(
        _dispatch_body,
        mesh=mesh,
        out_type=[
            jax.ShapeDtypeStruct((SLOTS, D_MODEL), jnp.float32),
            jax.ShapeDtypeStruct((SLOTS, SBW), jnp.float32),
        ],
        scratch_types=[
            pltpu.VMEM((TPW,), jnp.int32),
            pltpu.VMEM((TPW,), jnp.int32),
            pltpu.VMEM((TPW, D_MODEL), jnp.float32),
            pltpu.VMEM((TPW, SBW), jnp.float32),
            pltpu.VMEM((TPW, SBW), jnp.float32),
            pltpu.SemaphoreType.DMA,
        ],
    )
    return f(x2d, slots_t, sbc)


def _ffn_body(te_ref, tr_ref, xd_ref, w1_ref, b1_ref, w2_ref, b2_ref, s_ref,
              yd_ref):
    t = pl.program_id(0)
    rows = tr_ref[t]

    @pl.when(rows > 0)
    def _():
        xg = xd_ref[...].astype(jnp.bfloat16)
        h = jnp.dot(xg, w1_ref[0].astype(jnp.bfloat16),
                    preferred_element_type=jnp.float32) + b1_ref[0, 0]
        h = h * jax.nn.sigmoid(h)
        y = jnp.dot(h.astype(jnp.bfloat16), w2_ref[0].astype(jnp.bfloat16),
                    preferred_element_type=jnp.float32) + b2_ref[0, 0]
        yd_ref[...] = y * s_ref[:, 0:1]


def _ffn(xd, W1, b1r, W2, b2r, ssc, tile_expert, tile_rows):
    grid_spec = pltpu.PrefetchScalarGridSpec(
        num_scalar_prefetch=2,
        grid=(NT,),
        in_specs=[
            pl.BlockSpec((BT, D_MODEL), lambda t, te, tr: (t, 0)),
            pl.BlockSpec((1, D_MODEL, D_FF), lambda t, te, tr: (te[t], 0, 0)),
            pl.BlockSpec((1, 1, D_FF), lambda t, te, tr: (te[t], 0, 0)),
            pl.BlockSpec((1, D_FF, D_MODEL), lambda t, te, tr: (te[t], 0, 0)),
            pl.BlockSpec((1, 1, D_MODEL), lambda t, te, tr: (te[t], 0, 0)),
            pl.BlockSpec((BT, SBW), lambda t, te, tr: (t, 0)),
        ],
        out_specs=pl.BlockSpec((BT, D_MODEL), lambda t, te, tr: (t, 0)),
    )
    return pl.pallas_call(
        _ffn_body,
        grid_spec=grid_spec,
        out_shape=jax.ShapeDtypeStruct((SLOTS, D_MODEL), jnp.float32),
    )(tile_expert, tile_rows, xd, W1, b1r, W2, b2r, ssc)


def _combine_body(yd_hbm, slots_hbm, out_hbm, ia_v, ib_v, ra_v, rb_v,
                  sem_a, sem_b):
    core = lax.axis_index("c")
    sub = lax.axis_index("s")
    wid = sub * 2 + core
    t0 = pl.multiple_of(wid * TPW, TPW)
    pltpu.sync_copy(slots_hbm.at[0, pl.ds(t0, TPW)], ia_v)
    pltpu.sync_copy(slots_hbm.at[1, pl.ds(t0, TPW)], ib_v)
    ca = pltpu.async_copy(yd_hbm.at[ia_v], ra_v, sem_a)
    cb = pltpu.async_copy(yd_hbm.at[ib_v], rb_v, sem_b)
    ca.wait()
    cb.wait()
    for r in range(TPW):
        def _add(c, _, r=r):
            ra_v[r, pl.ds(c * 16, 16)] = (ra_v[r, pl.ds(c * 16, 16)]
                                          + rb_v[r, pl.ds(c * 16, 16)])
            return 0
        lax.fori_loop(0, D_MODEL // 16, _add, 0)
    pltpu.sync_copy(ra_v, out_hbm.at[pl.ds(t0, TPW)])


def _combine(yd, slots_t):
    mesh = plsc.VectorSubcoreMesh(core_axis_name="c", subcore_axis_name="s")
    f = pl.kernel(
        _combine_body,
        mesh=mesh,
        out_type=jax.ShapeDtypeStruct((N_TOKENS, D_MODEL), jnp.float32),
        scratch_types=[
            pltpu.VMEM((TPW,), jnp.int32),
            pltpu.VMEM((TPW,), jnp.int32),
            pltpu.VMEM((TPW, D_MODEL), jnp.float32),
            pltpu.VMEM((TPW, D_MODEL), jnp.float32),
            pltpu.SemaphoreType.DMA,
            pltpu.SemaphoreType.DMA,
        ],
    )
    return f(yd, slots_t)


def kernel(x, Wg, bg, W1, b1, W2, b2):
    x2d = x.reshape(-1, D_MODEL)
    bg2 = bg.reshape(1, N_EXPERTS)
    b1r = b1.reshape(N_EXPERTS, 1, D_FF)
    b2r = b2.reshape(N_EXPERTS, 1, D_MODEL)

    slots, counts, sbc = _gate(x2d, Wg, bg2)
    slots_t = slots.T
    counts = counts.reshape(N_EXPERTS)

    ntiles = (counts + BT - 1) // BT
    padded = ntiles * BT
    starts = jnp.concatenate([jnp.zeros((1,), jnp.int32),
                              jnp.cumsum(padded)[:-1].astype(jnp.int32)])
    tile_expert = jnp.repeat(jnp.arange(N_EXPERTS, dtype=jnp.int32), ntiles,
                             total_repeat_length=NT)
    tstart = jnp.arange(NT, dtype=jnp.int32) * BT
    local = tstart - starts[tile_expert]
    tile_rows = jnp.clip(counts[tile_expert] - local, 0, BT).astype(jnp.int32)

    xd, ssc = _dispatch(x2d, slots_t, sbc)
    yd = _ffn(xd, W1, b1r, W2, b2r, ssc, tile_expert, tile_rows)
    out = _combine(yd, slots_t)
    return out.reshape(x.shape)


# speedup vs baseline: 1.2096x; 1.0114x over previous
import jax
import jax.numpy as jnp
from jax import lax
from jax.experimental import pallas as pl
from jax.experimental.pallas import tpu as pltpu
from jax.experimental.pallas import tpu_sc as plsc

D_MODEL = 768
D_FF = 3072
N_EXPERTS = 8
N_TOKENS = 2048

BT = 512
SLOTS = ((2 * N_TOKENS + N_EXPERTS * (BT - 1) + BT - 1) // BT) * BT
NT = SLOTS // BT
SBW = 128


def _cumsum_rows(c, n):
    k = 1
    while k < n:
        shifted = jnp.concatenate(
            [jnp.zeros((k, c.shape[1]), c.dtype), c[: n - k]], axis=0)
        c = c + shifted
        k *= 2
    return c


def _gate_body(x_ref, wg_ref, bg_ref, slots_ref, counts_ref, sbc_ref):
    logits = jnp.dot(x_ref[...], wg_ref[...],
                     preferred_element_type=jnp.float32) + bg_ref[0]
    m = jnp.max(logits, axis=1, keepdims=True)
    p = jnp.exp(logits - m)
    probs = p / jnp.sum(p, axis=1, keepdims=True)
    iota = jax.lax.broadcasted_iota(jnp.int32, probs.shape, 1)
    m1 = jnp.max(probs, axis=1, keepdims=True)
    e1 = jnp.min(jnp.where(probs == m1, iota, N_EXPERTS), axis=1, keepdims=True)
    p2 = jnp.where(iota == e1, -jnp.inf, probs)
    m2 = jnp.max(p2, axis=1, keepdims=True)
    e2 = jnp.min(jnp.where(p2 == m2, iota, N_EXPERTS), axis=1, keepdims=True)

    oh1 = (iota == e1).astype(jnp.int32)
    oh2 = (iota == e2).astype(jnp.int32)
    c12 = _cumsum_rows(jnp.concatenate([oh1, oh2], axis=1), N_TOKENS)
    c1 = c12[:, :N_EXPERTS]
    c2 = c12[:, N_EXPERTS:]
    tot1 = c1[N_TOKENS - 1:N_TOKENS]
    counts = tot1 + c2[N_TOKENS - 1:N_TOKENS]
    rank1 = jnp.sum((c1 - 1) * oh1, axis=1, keepdims=True)
    rank2 = jnp.sum((c2 - 1 + tot1) * oh2, axis=1, keepdims=True)

    padded = lax.shift_left(
        lax.shift_right_logical(counts + (BT - 1), BT.bit_length() - 1),
        BT.bit_length() - 1)
    inc = padded
    k = 1
    while k < N_EXPERTS:
        inc = inc + jnp.concatenate(
            [jnp.zeros((1, k), jnp.int32), inc[:, : N_EXPERTS - k]], axis=1)
        k *= 2
    bases = inc - padded

    slot1 = rank1 + jnp.sum(oh1 * bases, axis=1, keepdims=True)
    slot2 = rank2 + jnp.sum(oh2 * bases, axis=1, keepdims=True)
    slots_ref[:, 0:1] = slot1
    slots_ref[:, 1:2] = slot2
    counts_ref[...] = counts
    ones = jnp.ones((1, SBW), jnp.float32)
    sbc_ref[0] = m1 * ones
    sbc_ref[1] = m2 * ones


def _gate(x2d, Wg, bg2):
    return pl.pallas_call(
        _gate_body,
        out_shape=[
            jax.ShapeDtypeStruct((N_TOKENS, 2), jnp.int32),
            jax.ShapeDtypeStruct((1, N_EXPERTS), jnp.int32),
            jax.ShapeDtypeStruct((2, N_TOKENS, SBW), jnp.float32),
        ],
    )(x2d, Wg, bg2)


TPW = N_TOKENS // 32


def _dispatch_body(x_hbm, slots_hbm, sbc_hbm, xd_hbm, ssc_hbm,
                   ia_v, ib_v, rows_v, sa_v, sb_v, sem):
    core = lax.axis_index("c")
    sub = lax.axis_index("s")
    wid = sub * 2 + core
    t0 = pl.multiple_of(wid * TPW, TPW)
    pltpu.sync_copy(slots_hbm.at[0, pl.ds(t0, TPW)], ia_v)
    pltpu.sync_copy(slots_hbm.at[1, pl.ds(t0, TPW)], ib_v)
    pltpu.sync_copy(sbc_hbm.at[0, pl.ds(t0, TPW)], sa_v)
    pltpu.sync_copy(sbc_hbm.at[1, pl.ds(t0, TPW)], sb_v)
    pltpu.sync_copy(x_hbm.at[pl.ds(t0, TPW)], rows_v)
    c1 = pltpu.async_copy(rows_v, xd_hbm.at[ia_v], sem)
    c2 = pltpu.async_copy(rows_v, xd_hbm.at[ib_v], sem)
    c3 = pltpu.async_copy(sa_v, ssc_hbm.at[ia_v], sem)
    c4 = pltpu.async_copy(sb_v, ssc_hbm.at[ib_v], sem)
    c1.wait()
    c2.wait()
    c3.wait()
    c4.wait()


def _dispatch(x2d, slots_t, sbc):
    mesh = plsc.VectorSubcoreMesh(core_axis_name="c", subcore_axis_name="s")
    f = pl.kernel(
        _dispatch_body,
        mesh=mesh,
        out_type=[
            jax.ShapeDtypeStruct((SLOTS, D_MODEL), jnp.float32),
            jax.ShapeDtypeStruct((SLOTS, SBW), jnp.float32),
        ],
        scratch_types=[
            pltpu.VMEM((TPW,), jnp.int32),
            pltpu.VMEM((TPW,), jnp.int32),
            pltpu.VMEM((TPW, D_MODEL), jnp.float32),
            pltpu.VMEM((TPW, SBW), jnp.float32),
            pltpu.VMEM((TPW, SBW), jnp.float32),
            pltpu.SemaphoreType.DMA,
        ],
    )
    return f(x2d, slots_t, sbc)


def _ffn_body(te_ref, tr_ref, xd_ref, w1_ref, b1_ref, w2_ref, b2_ref, s_ref,
              yd_ref):
    t = pl.program_id(0)
    rows = tr_ref[t]

    @pl.when(rows > 0)
    def _():
        xg = xd_ref[...].astype(jnp.bfloat16)
        h = jnp.dot(xg, w1_ref[0].astype(jnp.bfloat16),
                    preferred_element_type=jnp.float32) + b1_ref[0, 0]
        h = h * jax.nn.sigmoid(h)
        y = jnp.dot(h.astype(jnp.bfloat16), w2_ref[0].astype(jnp.bfloat16),
                    preferred_element_type=jnp.float32) + b2_ref[0, 0]
        yd_ref[...] = y * s_ref[:, 0:1]


def _ffn(xd, W1, b1r, W2, b2r, ssc, tile_expert, tile_rows):
    grid_spec = pltpu.PrefetchScalarGridSpec(
        num_scalar_prefetch=2,
        grid=(NT,),
        in_specs=[
            pl.BlockSpec((BT, D_MODEL),
                         lambda t, te, tr: (jnp.where(tr[t] > 0, t, 0), 0)),
            pl.BlockSpec((1, D_MODEL, D_FF), lambda t, te, tr: (te[t], 0, 0)),
            pl.BlockSpec((1, 1, D_FF), lambda t, te, tr: (te[t], 0, 0)),
            pl.BlockSpec((1, D_FF, D_MODEL), lambda t, te, tr: (te[t], 0, 0)),
            pl.BlockSpec((1, 1, D_MODEL), lambda t, te, tr: (te[t], 0, 0)),
            pl.BlockSpec((BT, SBW),
                         lambda t, te, tr: (jnp.where(tr[t] > 0, t, 0), 0)),
        ],
        out_specs=pl.BlockSpec((BT, D_MODEL), lambda t, te, tr: (t, 0)),
    )
    return pl.pallas_call(
        _ffn_body,
        grid_spec=grid_spec,
        out_shape=jax.ShapeDtypeStruct((SLOTS, D_MODEL), jnp.float32),
    )(tile_expert, tile_rows, xd, W1, b1r, W2, b2r, ssc)


def _combine_body(yd_hbm, slots_hbm, out_hbm, ia_v, ib_v, ra_v, rb_v,
                  sem_a, sem_b):
    core = lax.axis_index("c")
    sub = lax.axis_index("s")
    wid = sub * 2 + core
    t0 = pl.multiple_of(wid * TPW, TPW)
    pltpu.sync_copy(slots_hbm.at[0, pl.ds(t0, TPW)], ia_v)
    pltpu.sync_copy(slots_hbm.at[1, pl.ds(t0, TPW)], ib_v)
    ca = pltpu.async_copy(yd_hbm.at[ia_v], ra_v, sem_a)
    cb = pltpu.async_copy(yd_hbm.at[ib_v], rb_v, sem_b)
    ca.wait()
    cb.wait()
    for r in range(TPW):
        def _add(c, _, r=r):
            ra_v[r, pl.ds(c * 16, 16)] = (ra_v[r, pl.ds(c * 16, 16)]
                                          + rb_v[r, pl.ds(c * 16, 16)])
            return 0
        lax.fori_loop(0, D_MODEL // 16, _add, 0)
    pltpu.sync_copy(ra_v, out_hbm.at[pl.ds(t0, TPW)])


def _combine(yd, slots_t):
    mesh = plsc.VectorSubcoreMesh(core_axis_name="c", subcore_axis_name="s")
    f = pl.kernel(
        _combine_body,
        mesh=mesh,
        out_type=jax.ShapeDtypeStruct((N_TOKENS, D_MODEL), jnp.float32),
        scratch_types=[
            pltpu.VMEM((TPW,), jnp.int32),
            pltpu.VMEM((TPW,), jnp.int32),
            pltpu.VMEM((TPW, D_MODEL), jnp.float32),
            pltpu.VMEM((TPW, D_MODEL), jnp.float32),
            pltpu.SemaphoreType.DMA,
            pltpu.SemaphoreType.DMA,
        ],
    )
    return f(yd, slots_t)


def kernel(x, Wg, bg, W1, b1, W2, b2):
    x2d = x.reshape(-1, D_MODEL)
    bg2 = bg.reshape(1, N_EXPERTS)
    b1r = b1.reshape(N_EXPERTS, 1, D_FF)
    b2r = b2.reshape(N_EXPERTS, 1, D_MODEL)

    slots, counts, sbc = _gate(x2d, Wg, bg2)
    slots_t = slots.T
    counts = counts.reshape(N_EXPERTS)

    ntiles = (counts + BT - 1) // BT
    padded = ntiles * BT
    starts = jnp.concatenate([jnp.zeros((1,), jnp.int32),
                              jnp.cumsum(padded)[:-1].astype(jnp.int32)])
    tile_expert = jnp.repeat(jnp.arange(N_EXPERTS, dtype=jnp.int32), ntiles,
                             total_repeat_length=NT)
    tstart = jnp.arange(NT, dtype=jnp.int32) * BT
    local = tstart - starts[tile_expert]
    tile_rows = jnp.clip(counts[tile_expert] - local, 0, BT).astype(jnp.int32)

    xd, ssc = _dispatch(x2d, slots_t, sbc)
    yd = _ffn(xd, W1, b1r, W2, b2r, ssc, tile_expert, tile_rows)
    out = _combine(yd, slots_t)
    return out.reshape(x.shape)


# speedup vs baseline: 1.3124x; 1.0849x over previous
import jax
import jax.numpy as jnp
from jax import lax
from jax.experimental import pallas as pl
from jax.experimental.pallas import tpu as pltpu
from jax.experimental.pallas import tpu_sc as plsc

D_MODEL = 768
D_FF = 3072
N_EXPERTS = 8
N_TOKENS = 2048

BT = 512
SLOTS = ((2 * N_TOKENS + N_EXPERTS * (BT - 1) + BT - 1) // BT) * BT
NT = SLOTS // BT
SBW = 128


def _cumsum_rows(c, n):
    k = 1
    while k < n:
        shifted = jnp.concatenate(
            [jnp.zeros((k, c.shape[1]), c.dtype), c[: n - k]], axis=0)
        c = c + shifted
        k *= 2
    return c


def _gate_body(x_ref, wg_ref, bg_ref, slots_ref, counts_ref, sbc_ref):
    logits = jnp.dot(x_ref[...], wg_ref[...],
                     preferred_element_type=jnp.float32) + bg_ref[0]
    m = jnp.max(logits, axis=1, keepdims=True)
    p = jnp.exp(logits - m)
    probs = p / jnp.sum(p, axis=1, keepdims=True)
    iota = jax.lax.broadcasted_iota(jnp.int32, probs.shape, 1)
    m1 = jnp.max(probs, axis=1, keepdims=True)
    e1 = jnp.min(jnp.where(probs == m1, iota, N_EXPERTS), axis=1, keepdims=True)
    p2 = jnp.where(iota == e1, -jnp.inf, probs)
    m2 = jnp.max(p2, axis=1, keepdims=True)
    e2 = jnp.min(jnp.where(p2 == m2, iota, N_EXPERTS), axis=1, keepdims=True)

    oh1 = (iota == e1).astype(jnp.int32)
    oh2 = (iota == e2).astype(jnp.int32)
    c12 = _cumsum_rows(jnp.concatenate([oh1, oh2], axis=1), N_TOKENS)
    c1 = c12[:, :N_EXPERTS]
    c2 = c12[:, N_EXPERTS:]
    tot1 = c1[N_TOKENS - 1:N_TOKENS]
    counts = tot1 + c2[N_TOKENS - 1:N_TOKENS]
    rank1 = jnp.sum((c1 - 1) * oh1, axis=1, keepdims=True)
    rank2 = jnp.sum((c2 - 1 + tot1) * oh2, axis=1, keepdims=True)

    padded = lax.shift_left(
        lax.shift_right_logical(counts + (BT - 1), BT.bit_length() - 1),
        BT.bit_length() - 1)
    inc = padded
    k = 1
    while k < N_EXPERTS:
        inc = inc + jnp.concatenate(
            [jnp.zeros((1, k), jnp.int32), inc[:, : N_EXPERTS - k]], axis=1)
        k *= 2
    bases = inc - padded

    slot1 = rank1 + jnp.sum(oh1 * bases, axis=1, keepdims=True)
    slot2 = rank2 + jnp.sum(oh2 * bases, axis=1, keepdims=True)
    slots_ref[:, 0:1] = slot1
    slots_ref[:, 1:2] = slot2
    counts_ref[...] = counts
    ones = jnp.ones((1, SBW), jnp.float32)
    sbc_ref[0] = m1 * ones
    sbc_ref[1] = m2 * ones


def _gate(x2d, Wg, bg2):
    return pl.pallas_call(
        _gate_body,
        out_shape=[
            jax.ShapeDtypeStruct((N_TOKENS, 2), jnp.int32),
            jax.ShapeDtypeStruct((1, N_EXPERTS), jnp.int32),
            jax.ShapeDtypeStruct((2, N_TOKENS, SBW), jnp.float32),
        ],
    )(x2d, Wg, bg2)


TPW = N_TOKENS // 32


def _dispatch_body(x_hbm, slots_hbm, sbc_hbm, xd_hbm, ssc_hbm,
                   ia_v, ib_v, rows_v, sa_v, sb_v, sem):
    core = lax.axis_index("c")
    sub = lax.axis_index("s")
    wid = sub * 2 + core
    t0 = pl.multiple_of(wid * TPW, TPW)
    pltpu.sync_copy(slots_hbm.at[0, pl.ds(t0, TPW)], ia_v)
    pltpu.sync_copy(slots_hbm.at[1, pl.ds(t0, TPW)], ib_v)
    pltpu.sync_copy(sbc_hbm.at[0, pl.ds(t0, TPW)], sa_v)
    pltpu.sync_copy(sbc_hbm.at[1, pl.ds(t0, TPW)], sb_v)
    pltpu.sync_copy(x_hbm.at[pl.ds(t0, TPW)], rows_v)
    c1 = pltpu.async_copy(rows_v, xd_hbm.at[ia_v], sem)
    c2 = pltpu.async_copy(rows_v, xd_hbm.at[ib_v], sem)
    c3 = pltpu.async_copy(sa_v, ssc_hbm.at[ia_v], sem)
    c4 = pltpu.async_copy(sb_v, ssc_hbm.at[ib_v], sem)
    c1.wait()
    c2.wait()
    c3.wait()
    c4.wait()


def _dispatch(x2d, slots_t, sbc):
    mesh = plsc.VectorSubcoreMesh(core_axis_name="c", subcore_axis_name="s")
    f = pl.kernel(
        _dispatch_body,
        mesh=mesh,
        out_type=[
            jax.ShapeDtypeStruct((SLOTS, D_MODEL), jnp.float32),
            jax.ShapeDtypeStruct((SLOTS, SBW), jnp.float32),
        ],
        scratch_types=[
            pltpu.VMEM((TPW,), jnp.int32),
            pltpu.VMEM((TPW,), jnp.int32),
            pltpu.VMEM((TPW, D_MODEL), jnp.float32),
            pltpu.VMEM((TPW, SBW), jnp.float32),
            pltpu.VMEM((TPW, SBW), jnp.float32),
            pltpu.SemaphoreType.DMA,
        ],
    )
    return f(x2d, slots_t, sbc)


def _ffn_body(te_ref, tr_ref, xd_ref, w1_ref, b1_ref, w2_ref, b2_ref, s_ref,
              yd_ref):
    t = pl.program_id(0)
    rows = tr_ref[t]

    @pl.when(rows > 0)
    def _():
        xg = xd_ref[...].astype(jnp.bfloat16)
        y = b2_ref[0, 0]
        nfc = 4
        fb = D_FF // nfc
        for f in range(nfc):
            h = jnp.dot(xg, w1_ref[0][:, f * fb:(f + 1) * fb].astype(
                jnp.bfloat16), preferred_element_type=jnp.float32)
            h = h + b1_ref[0, 0, f * fb:(f + 1) * fb]
            h = h * jax.nn.sigmoid(h)
            y = y + jnp.dot(h.astype(jnp.bfloat16),
                            w2_ref[0][f * fb:(f + 1) * fb].astype(jnp.bfloat16),
                            preferred_element_type=jnp.float32)
        yd_ref[...] = y * s_ref[:, 0:1]


def _ffn(xd, W1, b1r, W2, b2r, ssc, tile_expert, tile_rows):
    grid_spec = pltpu.PrefetchScalarGridSpec(
        num_scalar_prefetch=2,
        grid=(NT,),
        in_specs=[
            pl.BlockSpec((BT, D_MODEL),
                         lambda t, te, tr: (jnp.where(tr[t] > 0, t, 0), 0)),
            pl.BlockSpec((1, D_MODEL, D_FF), lambda t, te, tr: (te[t], 0, 0)),
            pl.BlockSpec((1, 1, D_FF), lambda t, te, tr: (te[t], 0, 0)),
            pl.BlockSpec((1, D_FF, D_MODEL), lambda t, te, tr: (te[t], 0, 0)),
            pl.BlockSpec((1, 1, D_MODEL), lambda t, te, tr: (te[t], 0, 0)),
            pl.BlockSpec((BT, SBW),
                         lambda t, te, tr: (jnp.where(tr[t] > 0, t, 0), 0)),
        ],
        out_specs=pl.BlockSpec((BT, D_MODEL), lambda t, te, tr: (t, 0)),
    )
    return pl.pallas_call(
        _ffn_body,
        grid_spec=grid_spec,
        out_shape=jax.ShapeDtypeStruct((SLOTS, D_MODEL), jnp.float32),
    )(tile_expert, tile_rows, xd, W1, b1r, W2, b2r, ssc)


def _combine_body(yd_hbm, slots_hbm, out_hbm, ia_v, ib_v, ra_v, rb_v,
                  sem_a, sem_b):
    core = lax.axis_index("c")
    sub = lax.axis_index("s")
    wid = sub * 2 + core
    t0 = pl.multiple_of(wid * TPW, TPW)
    pltpu.sync_copy(slots_hbm.at[0, pl.ds(t0, TPW)], ia_v)
    pltpu.sync_copy(slots_hbm.at[1, pl.ds(t0, TPW)], ib_v)
    ca = pltpu.async_copy(yd_hbm.at[ia_v], ra_v, sem_a)
    cb = pltpu.async_copy(yd_hbm.at[ib_v], rb_v, sem_b)
    ca.wait()
    cb.wait()

    def _add_row(r, _):
        for c in range(D_MODEL // 16):
            ra_v[r, pl.ds(c * 16, 16)] = (ra_v[r, pl.ds(c * 16, 16)]
                                          + rb_v[r, pl.ds(c * 16, 16)])
        return 0

    lax.fori_loop(0, TPW, _add_row, 0)
    pltpu.sync_copy(ra_v, out_hbm.at[pl.ds(t0, TPW)])


def _combine(yd, slots_t):
    mesh = plsc.VectorSubcoreMesh(core_axis_name="c", subcore_axis_name="s")
    f = pl.kernel(
        _combine_body,
        mesh=mesh,
        out_type=jax.ShapeDtypeStruct((N_TOKENS, D_MODEL), jnp.float32),
        scratch_types=[
            pltpu.VMEM((TPW,), jnp.int32),
            pltpu.VMEM((TPW,), jnp.int32),
            pltpu.VMEM((TPW, D_MODEL), jnp.float32),
            pltpu.VMEM((TPW, D_MODEL), jnp.float32),
            pltpu.SemaphoreType.DMA,
            pltpu.SemaphoreType.DMA,
        ],
    )
    return f(yd, slots_t)


def kernel(x, Wg, bg, W1, b1, W2, b2):
    x2d = x.reshape(-1, D_MODEL)
    bg2 = bg.reshape(1, N_EXPERTS)
    b1r = b1.reshape(N_EXPERTS, 1, D_FF)
    b2r = b2.reshape(N_EXPERTS, 1, D_MODEL)

    slots, counts, sbc = _gate(x2d, Wg, bg2)
    slots_t = slots.T
    counts = counts.reshape(N_EXPERTS)

    ntiles = (counts + BT - 1) // BT
    padded = ntiles * BT
    starts = jnp.concatenate([jnp.zeros((1,), jnp.int32),
                              jnp.cumsum(padded)[:-1].astype(jnp.int32)])
    tile_expert = jnp.repeat(jnp.arange(N_EXPERTS, dtype=jnp.int32), ntiles,
                             total_repeat_length=NT)
    tstart = jnp.arange(NT, dtype=jnp.int32) * BT
    local = tstart - starts[tile_expert]
    tile_rows = jnp.clip(counts[tile_expert] - local, 0, BT).astype(jnp.int32)

    xd, ssc = _dispatch(x2d, slots_t, sbc)
    yd = _ffn(xd, W1, b1r, W2, b2r, ssc, tile_expert, tile_rows)
    out = _combine(yd, slots_t)
    return out.reshape(x.shape)
